# baseline (speedup 1.0000x reference)
import functools

import jax
import jax.numpy as jnp
from jax import lax
from jax.experimental import pallas as pl
from jax.experimental.pallas import tpu as pltpu
from jax.experimental.pallas import tpu_sc as plsc
from jax.scipy.linalg import block_diag


HP = 16
MLP_R = 10000


def _mlp_weights(atom_fea, W1, b1t, W2blk, b2t):
    N, D = atom_fea.shape
    HID = W1.shape[1]
    R, C = MLP_R, MLP_R // 8

    def body(x_ref, w1_ref, b1_ref, w2_ref, b2_ref, o_ref):
        w1 = w1_ref[...]
        zs = [
            jnp.dot(x_ref[pl.ds(g * C, C), :], w1,
                    preferred_element_type=jnp.float32)
            for g in range(8)
        ]
        z = jnp.concatenate(zs, axis=1) + b1_ref[...]
        h = z * (1.0 / (1.0 + jnp.exp(-z)))
        lg = jnp.dot(h, w2_ref[...], preferred_element_type=jnp.float32)
        lgp = lg + b2_ref[...]
        lane = jax.lax.broadcasted_iota(jnp.int32, (C, 128), 1)
        valid = (lane & (HP - 1)) < 4
        a = jnp.maximum(lgp, pltpu.roll(lgp, 127, 1))
        b = jnp.maximum(a, pltpu.roll(a, 126, 1))
        m = jnp.maximum(b, pltpu.roll(b, 4, 1))
        e = jnp.where(valid, jnp.exp(lgp - m), 0.0)
        a2 = e + pltpu.roll(e, 127, 1)
        b2 = a2 + pltpu.roll(a2, 126, 1)
        s = b2 + pltpu.roll(b2, 4, 1)
        o_ref[...] = (e / s).reshape(1, C, 128)

    return pl.pallas_call(
        body,
        grid=(N // R,),
        in_specs=[
            pl.BlockSpec((R, D), lambda i: (i, 0)),
            pl.BlockSpec((D, HID), lambda i: (0, 0)),
            pl.BlockSpec((1, 8 * HID), lambda i: (0, 0)),
            pl.BlockSpec((8 * HID, 8 * HP), lambda i: (0, 0)),
            pl.BlockSpec((1, 8 * HP), lambda i: (0, 0)),
        ],
        out_specs=pl.BlockSpec((1, C, 128), lambda i: (i, 0, 0)),
        out_shape=jax.ShapeDtypeStruct((N // R, C, 128), jnp.float32),
    )(atom_fea, W1, b1t, W2blk, b2t)


def _sc_pool(atom_fea, cw, idx, idxw, H):
    N, D = atom_fea.shape
    B, A = idx.shape
    info = plsc.get_sparse_core_info()
    NC, NS, L = info.num_cores, info.num_subcores, info.num_lanes
    NW = NC * NS
    per_w = B // NW
    nseg = D // L
    mesh = plsc.VectorSubcoreMesh(core_axis_name="c", subcore_axis_name="s")

    @functools.partial(
        pl.kernel,
        mesh=mesh,
        compiler_params=pltpu.CompilerParams(use_tc_tiling_on_sc=False),
        out_type=jax.ShapeDtypeStruct((B, H * D), jnp.float32),
        scratch_types=[
            pltpu.VMEM((per_w, A), jnp.int32),
            pltpu.VMEM((per_w, A), jnp.int32),
            pltpu.VMEM((A, D), jnp.float32),
            pltpu.VMEM((A, D), jnp.float32),
            pltpu.VMEM((A, HP), jnp.float32),
            pltpu.VMEM((A, HP), jnp.float32),
            pltpu.VMEM((H * D,), jnp.float32),
            pltpu.SemaphoreType.DMA,
            pltpu.SemaphoreType.DMA,
            pltpu.SemaphoreType.DMA,
            pltpu.SemaphoreType.DMA,
        ],
    )
    def pool(atom_hbm, cw_hbm, idx_hbm, idxw_hbm, out_hbm,
             idx_v, idxw_v, rows0, rows1, cwr0, cwr1, out_v,
             sa0, sa1, sw0, sw1):
        wid = lax.axis_index("s") * NC + lax.axis_index("c")
        base = wid * per_w
        pltpu.sync_copy(idx_hbm.at[pl.ds(base, per_w)], idx_v)
        pltpu.sync_copy(idxw_hbm.at[pl.ds(base, per_w)], idxw_v)
        slots = ((rows0, cwr0, sa0, sw0), (rows1, cwr1, sa1, sw1))

        def start(i, rows_v, cwr_v, sa, sw):
            pltpu.async_copy(atom_hbm.at[idx_v.at[i]], rows_v, sa)
            pltpu.async_copy(cw_hbm.at[idxw_v.at[i]], cwr_v, sw)

        def wait(i, rows_v, cwr_v, sa, sw):
            pltpu.make_async_copy(atom_hbm.at[idx_v.at[i]], rows_v, sa).wait()
            pltpu.make_async_copy(cw_hbm.at[idxw_v.at[i]], cwr_v, sw).wait()

        HREG = H // 2
        zero = jnp.zeros((L,), jnp.float32)

        def compute(i, rows_v, cwr_v, *_):
            for j in range(HREG * nseg, H * nseg):
                out_v[pl.ds(j * L, L)] = zero

            def atom(a, accs):
                accs = list(accs)
                cwvec = cwr_v[a, :]
                dnums = lax.GatherDimensionNumbers(
                    offset_dims=(), collapsed_slice_dims=(0,),
                    start_index_map=(0,))
                cwb = [
                    lax.gather(cwvec, jnp.full((L, 1), h, jnp.int32), dnums,
                               slice_sizes=(1,),
                               mode=lax.GatherScatterMode.PROMISE_IN_BOUNDS)
                    for h in range(H)
                ]
                for seg in range(nseg):
                    v = rows_v[a, pl.ds(seg * L, L)]
                    for h in range(HREG):
                        accs[h * nseg + seg] = accs[h * nseg + seg] + cwb[h] * v
                    for h in range(HREG, H):
                        plsc.addupdate(
                            out_v.at[pl.ds((h * nseg + seg) * L, L)],
                            cwb[h] * v)
                return tuple(accs)

            accs = lax.fori_loop(
                0, A, atom,
                tuple(zero for _ in range(HREG * nseg)))
            for j in range(HREG * nseg):
                out_v[pl.ds(j * L, L)] = accs[j]
            pltpu.sync_copy(out_v, out_hbm.at[base + i])

        start(0, *slots[0])

        def body(j, carry):
            i0 = 2 * j
            i1 = 2 * j + 1
            start(i1, *slots[1])
            wait(i0, *slots[0])
            compute(i0, *slots[0])

            @pl.when(i1 + 1 < per_w)
            def _():
                start(i1 + 1, *slots[0])

            wait(i1, *slots[1])
            compute(i1, *slots[1])
            return carry

        lax.fori_loop(0, per_w // 2, body, 0)

    return pool(atom_fea, cw, idx, idxw)


def _project(flat, Wp, bp_2d):
    B, HD = flat.shape
    D = Wp.shape[1]

    def body(f_ref, wp_ref, bp_ref, o_ref):
        y = jnp.dot(f_ref[...], wp_ref[...], preferred_element_type=jnp.float32)
        y = y + bp_ref[...]
        o_ref[...] = y * (1.0 / (1.0 + jnp.exp(-y)))

    return pl.pallas_call(
        body,
        out_shape=jax.ShapeDtypeStruct((B, D), jnp.float32),
    )(flat, Wp, bp_2d)


def kernel(atom_fea, crystal_atom_idx, W1, b1, W2, b2, Wp, bp):
    N = atom_fea.shape[0]
    H = W2.shape[1]
    W2p = jnp.pad(W2, ((0, 0), (0, HP - H)))
    b2p = jnp.concatenate([b2, jnp.full((HP - H,), -1e30, b2.dtype)])
    W2blk = block_diag(*([W2p] * 8))
    b1t = jnp.tile(b1, 8).reshape(1, -1)
    b2t = jnp.tile(b2p, 8).reshape(1, -1)
    cw_packed = _mlp_weights(atom_fea, W1, b1t, W2blk, b2t)
    cw16 = cw_packed.reshape(N, HP)
    rem = crystal_atom_idx % MLP_R
    idxw = (crystal_atom_idx - rem) + 8 * (rem % (MLP_R // 8)) \
        + rem // (MLP_R // 8)
    flat = _sc_pool(atom_fea, cw16, crystal_atom_idx, idxw, H)
    return _project(flat, Wp, bp.reshape(1, -1))


# speedup vs baseline: 1.7847x; 1.7847x over previous
import functools

import jax
import jax.numpy as jnp
from jax import lax
from jax.experimental import pallas as pl
from jax.experimental.pallas import tpu as pltpu
from jax.experimental.pallas import tpu_sc as plsc
from jax.scipy.linalg import block_diag


HP = 16
MLP_R = 20000


def _mlp_weights(atom_fea, W1, b1t, W2blk, b2t):
    N, D = atom_fea.shape
    HID = W1.shape[1]
    R, C = MLP_R, MLP_R // 8

    def body(x_ref, w1_ref, b1_ref, w2_ref, b2_ref, o_ref):
        w1 = w1_ref[...]
        zs = [
            jnp.dot(x_ref[pl.ds(g * C, C), :], w1,
                    preferred_element_type=jnp.float32)
            for g in range(8)
        ]
        z = jnp.concatenate(zs, axis=1) + b1_ref[...]
        h = z * (1.0 / (1.0 + jnp.exp(-z)))
        lg = jnp.dot(h, w2_ref[...], preferred_element_type=jnp.float32)
        lgp = lg + b2_ref[...]
        lane = jax.lax.broadcasted_iota(jnp.int32, (C, 128), 1)
        valid = (lane & (HP - 1)) < 4
        a = jnp.maximum(lgp, pltpu.roll(lgp, 127, 1))
        b = jnp.maximum(a, pltpu.roll(a, 126, 1))
        m = jnp.maximum(b, pltpu.roll(b, 4, 1))
        e = jnp.where(valid, jnp.exp(lgp - m), 0.0)
        a2 = e + pltpu.roll(e, 127, 1)
        b2 = a2 + pltpu.roll(a2, 126, 1)
        s = b2 + pltpu.roll(b2, 4, 1)
        o_ref[...] = (e / s).reshape(1, C, 128)

    return pl.pallas_call(
        body,
        grid=(N // R,),
        in_specs=[
            pl.BlockSpec((R, D), lambda i: (i, 0)),
            pl.BlockSpec((D, HID), lambda i: (0, 0)),
            pl.BlockSpec((1, 8 * HID), lambda i: (0, 0)),
            pl.BlockSpec((8 * HID, 8 * HP), lambda i: (0, 0)),
            pl.BlockSpec((1, 8 * HP), lambda i: (0, 0)),
        ],
        out_specs=pl.BlockSpec((1, C, 128), lambda i: (i, 0, 0)),
        out_shape=jax.ShapeDtypeStruct((N // R, C, 128), jnp.float32),
    )(atom_fea, W1, b1t, W2blk, b2t)


def _sc_pool(atom_fea, cw, idx, idxw, H):
    N, D = atom_fea.shape
    B, A = idx.shape
    info = plsc.get_sparse_core_info()
    NC, NS, L = info.num_cores, info.num_subcores, info.num_lanes
    NW = NC * NS
    per_w = B // NW
    nseg = D // L
    mesh = plsc.VectorSubcoreMesh(core_axis_name="c", subcore_axis_name="s")

    @functools.partial(
        pl.kernel,
        mesh=mesh,
        compiler_params=pltpu.CompilerParams(use_tc_tiling_on_sc=False),
        out_type=jax.ShapeDtypeStruct((B, H * D), jnp.float32),
        scratch_types=[
            pltpu.VMEM((per_w, A), jnp.int32),
            pltpu.VMEM((per_w, A), jnp.int32),
            pltpu.VMEM((A, D), jnp.float32),
            pltpu.VMEM((A, D), jnp.float32),
            pltpu.VMEM((A, HP), jnp.float32),
            pltpu.VMEM((A, HP), jnp.float32),
            pltpu.VMEM((H * D,), jnp.float32),
            pltpu.SemaphoreType.DMA,
            pltpu.SemaphoreType.DMA,
            pltpu.SemaphoreType.DMA,
            pltpu.SemaphoreType.DMA,
        ],
    )
    def pool(atom_hbm, cw_hbm, idx_hbm, idxw_hbm, out_hbm,
             idx_v, idxw_v, rows0, rows1, cwr0, cwr1, out_v,
             sa0, sa1, sw0, sw1):
        wid = lax.axis_index("s") * NC + lax.axis_index("c")
        base = wid * per_w
        pltpu.sync_copy(idx_hbm.at[pl.ds(base, per_w)], idx_v)
        pltpu.sync_copy(idxw_hbm.at[pl.ds(base, per_w)], idxw_v)
        slots = ((rows0, cwr0, sa0, sw0), (rows1, cwr1, sa1, sw1))

        def start(i, rows_v, cwr_v, sa, sw):
            pltpu.async_copy(atom_hbm.at[idx_v.at[i]], rows_v, sa)
            pltpu.async_copy(cw_hbm.at[idxw_v.at[i]], cwr_v, sw)

        def wait(i, rows_v, cwr_v, sa, sw):
            pltpu.make_async_copy(atom_hbm.at[idx_v.at[i]], rows_v, sa).wait()
            pltpu.make_async_copy(cw_hbm.at[idxw_v.at[i]], cwr_v, sw).wait()

        def compute(i, rows_v, cwr_v, *_):
            def atom(a, accs):
                accs = list(accs)
                cwvec = cwr_v[a, :]
                dnums = lax.GatherDimensionNumbers(
                    offset_dims=(), collapsed_slice_dims=(0,),
                    start_index_map=(0,))
                cwb = [
                    lax.gather(cwvec, jnp.full((L, 1), h, jnp.int32), dnums,
                               slice_sizes=(1,),
                               mode=lax.GatherScatterMode.PROMISE_IN_BOUNDS)
                    for h in range(H)
                ]
                for seg in range(nseg):
                    v = rows_v[a, pl.ds(seg * L, L)]
                    for h in range(H):
                        accs[h * nseg + seg] = accs[h * nseg + seg] + cwb[h] * v
                return tuple(accs)

            accs = lax.fori_loop(
                0, A, atom,
                tuple(jnp.zeros((L,), jnp.float32) for _ in range(H * nseg)))
            for j in range(H * nseg):
                out_v[pl.ds(j * L, L)] = accs[j]
            pltpu.sync_copy(out_v, out_hbm.at[base + i])

        start(0, *slots[0])

        def body(j, carry):
            i0 = 2 * j
            i1 = 2 * j + 1
            start(i1, *slots[1])
            wait(i0, *slots[0])
            compute(i0, *slots[0])

            @pl.when(i1 + 1 < per_w)
            def _():
                start(i1 + 1, *slots[0])

            wait(i1, *slots[1])
            compute(i1, *slots[1])
            return carry

        lax.fori_loop(0, per_w // 2, body, 0)

    return pool(atom_fea, cw, idx, idxw)


def _project(flat, Wp, bp_2d):
    B, HD = flat.shape
    D = Wp.shape[1]

    def body(f_ref, wp_ref, bp_ref, o_ref):
        y = jnp.dot(f_ref[...], wp_ref[...], preferred_element_type=jnp.float32)
        y = y + bp_ref[...]
        o_ref[...] = y * (1.0 / (1.0 + jnp.exp(-y)))

    return pl.pallas_call(
        body,
        out_shape=jax.ShapeDtypeStruct((B, D), jnp.float32),
    )(flat, Wp, bp_2d)


def kernel(atom_fea, crystal_atom_idx, W1, b1, W2, b2, Wp, bp):
    N = atom_fea.shape[0]
    H = W2.shape[1]
    W2p = jnp.pad(W2, ((0, 0), (0, HP - H)))
    b2p = jnp.concatenate([b2, jnp.full((HP - H,), -1e30, b2.dtype)])
    W2blk = block_diag(*([W2p] * 8))
    b1t = jnp.tile(b1, 8).reshape(1, -1)
    b2t = jnp.tile(b2p, 8).reshape(1, -1)
    cw_packed = _mlp_weights(atom_fea, W1, b1t, W2blk, b2t)
    cw16 = cw_packed.reshape(N, HP)
    rem = crystal_atom_idx % MLP_R
    idxw = (crystal_atom_idx - rem) + 8 * (rem % (MLP_R // 8)) \
        + rem // (MLP_R // 8)
    flat = _sc_pool(atom_fea, cw16, crystal_atom_idx, idxw, H)
    return _project(flat, Wp, bp.reshape(1, -1))
